# Initial kernel scaffold; baseline (speedup 1.0000x reference)
import jax
import jax.numpy as jnp
from jax.experimental import pallas as pl


def kernel(x, edge_index, edge_attr, self_loop_index, self_loop_type, W_enc0, b_enc0, W1_0, b1_0, gamma0, beta0, W2_0, b2_0, W_enc1, b_enc1, W1_1, b1_1, gamma1, beta1, W2_1, b2_1):
    raise NotImplementedError("write your pallas kernel here")


# speedup vs baseline: 6.4885x; 6.4885x over previous
import functools

import jax
import jax.numpy as jnp
import numpy as np
from jax import lax
from jax.experimental import pallas as pl
from jax.experimental.pallas import tpu as pltpu
from jax.experimental.pallas import tpu_sc as plsc

N = 10000
E = 320000
D = 128
DE = 16
EPS = 1e-05

NC = 2
NS = 16
NW = NC * NS
EPW = E // NW
K = 80
NCHUNK = EPW // K
RO = 624
ZR = 16
TAIL = N - NS * RO

_MESH = plsc.VectorSubcoreMesh(core_axis_name="c", subcore_axis_name="s")


def _make_spmm(with_sd: bool):
    out_type = [jax.ShapeDtypeStruct((NC, N, D), jnp.float32)]
    if with_sd:
        out_type += [jax.ShapeDtypeStruct((NC, N, DE), jnp.float32),
                     jax.ShapeDtypeStruct((NC, N, DE), jnp.float32)]
    scratch = [
        pltpu.VMEM((K,), jnp.int32),
        pltpu.VMEM((K,), jnp.int32),
        pltpu.VMEM((K, D), jnp.float32),
        pltpu.VMEM((ZR, D), jnp.float32),
        pltpu.VMEM_SHARED((N, D), jnp.float32),
        pltpu.SemaphoreType.DMA,
    ]
    if with_sd:
        scratch += [
            pltpu.VMEM((K, DE), jnp.float32),
            pltpu.VMEM((K, DE), jnp.float32),
            pltpu.VMEM((ZR, DE), jnp.float32),
            pltpu.VMEM_SHARED((N, DE), jnp.float32),
            pltpu.VMEM_SHARED((N, DE), jnp.float32),
        ]

    def body(h_hbm, src_hbm, dst_hbm, ea_hbm, *refs):
        if with_sd:
            (agg_out, s_out, deg_out, srcv, dstv, rows, zbuf, agg_sh, sem,
             eav, ones, zbuf16, s_sh, deg_sh) = refs
        else:
            agg_out, srcv, dstv, rows, zbuf, agg_sh, sem = refs
        cid = lax.axis_index("c")
        sid = lax.axis_index("s")

        zv = jnp.zeros((16,), jnp.float32)

        @pl.loop(0, ZR)
        def _(i):
            @pl.loop(0, D // 16)
            def _(j):
                zbuf[i, pl.ds(j * 16, 16)] = zv

        if with_sd:
            ov = jnp.ones((16,), jnp.float32)

            @pl.loop(0, ZR)
            def _(i):
                zbuf16[i, pl.ds(0, 16)] = zv

            @pl.loop(0, K)
            def _(i):
                ones[i, pl.ds(0, 16)] = ov

        rbase = sid * RO
        for r in range(RO // ZR):
            pltpu.sync_copy(zbuf, agg_sh.at[pl.ds(rbase + r * ZR, ZR)])
            if with_sd:
                pltpu.sync_copy(zbuf16, s_sh.at[pl.ds(rbase + r * ZR, ZR)])
                pltpu.sync_copy(zbuf16, deg_sh.at[pl.ds(rbase + r * ZR, ZR)])

        @pl.when(sid == NS - 1)
        def _():
            pltpu.sync_copy(zbuf.at[pl.ds(0, TAIL)], agg_sh.at[pl.ds(NS * RO, TAIL)])
            if with_sd:
                pltpu.sync_copy(zbuf16.at[pl.ds(0, TAIL)], s_sh.at[pl.ds(NS * RO, TAIL)])
                pltpu.sync_copy(zbuf16.at[pl.ds(0, TAIL)], deg_sh.at[pl.ds(NS * RO, TAIL)])

        plsc.subcore_barrier()

        ebase = (sid * NC + cid) * EPW

        @pl.loop(0, NCHUNK)
        def _(j):
            off = ebase + j * K
            pltpu.sync_copy(src_hbm.at[pl.ds(off, K)], srcv)
            pltpu.sync_copy(dst_hbm.at[pl.ds(off, K)], dstv)
            if with_sd:
                pltpu.sync_copy(ea_hbm.at[pl.ds(off, K)], eav)
            pltpu.async_copy(h_hbm.at[srcv], rows, sem).wait()
            pltpu.sync_copy(rows, agg_sh.at[dstv], add=True)
            if with_sd:
                pltpu.sync_copy(eav, s_sh.at[dstv], add=True)
                pltpu.sync_copy(ones, deg_sh.at[dstv], add=True)

        plsc.subcore_barrier()
        pltpu.sync_copy(agg_sh.at[pl.ds(rbase, RO)],
                        agg_out.at[cid, pl.ds(rbase, RO)])
        if with_sd:
            pltpu.sync_copy(s_sh.at[pl.ds(rbase, RO)],
                            s_out.at[cid, pl.ds(rbase, RO)])
            pltpu.sync_copy(deg_sh.at[pl.ds(rbase, RO)],
                            deg_out.at[cid, pl.ds(rbase, RO)])

        @pl.when(sid == NS - 1)
        def _():
            pltpu.sync_copy(agg_sh.at[pl.ds(NS * RO, TAIL)],
                            agg_out.at[cid, pl.ds(NS * RO, TAIL)])
            if with_sd:
                pltpu.sync_copy(s_sh.at[pl.ds(NS * RO, TAIL)],
                                s_out.at[cid, pl.ds(NS * RO, TAIL)])
                pltpu.sync_copy(deg_sh.at[pl.ds(NS * RO, TAIL)],
                                deg_out.at[cid, pl.ds(NS * RO, TAIL)])

    return functools.partial(
        pl.kernel, mesh=_MESH, out_type=tuple(out_type), scratch_types=scratch,
        compiler_params=pltpu.CompilerParams(use_tc_tiling_on_sc=False))(body)


_spmm_sd = _make_spmm(True)
_spmm = _make_spmm(False)


def _make_mlp(final_relu: bool, with_sd_inputs: bool):
    R = 2000

    def body(agg_ref, h_ref, s_ref, d_ref, a_ref, b_ref, v_ref, u_ref,
             w2_ref, b2_ref, o_ref):
        z = agg_ref[0] + agg_ref[1] + h_ref[...]
        sarr = s_ref[0] + s_ref[1]
        darr = d_ref[0] + d_ref[1]
        dcol = darr[:, :1]
        pre = (jnp.dot(z, a_ref[...], preferred_element_type=jnp.float32)
               + jnp.dot(sarr, b_ref[...], preferred_element_type=jnp.float32)
               + dcol * v_ref[...] + u_ref[...])
        t = jnp.maximum(pre, 0.0)
        out = jnp.dot(t, w2_ref[...], preferred_element_type=jnp.float32) + b2_ref[...]
        if final_relu:
            out = jnp.maximum(out, 0.0)
        o_ref[...] = out

    grid = (N // R,)
    in_specs = [
        pl.BlockSpec((NC, R, D), lambda i: (0, i, 0)),
        pl.BlockSpec((R, D), lambda i: (i, 0)),
        pl.BlockSpec((NC, R, DE), lambda i: (0, i, 0)),
        pl.BlockSpec((NC, R, DE), lambda i: (0, i, 0)),
        pl.BlockSpec((D, 2 * D), lambda i: (0, 0)),
        pl.BlockSpec((DE, 2 * D), lambda i: (0, 0)),
        pl.BlockSpec((1, 2 * D), lambda i: (0, 0)),
        pl.BlockSpec((1, 2 * D), lambda i: (0, 0)),
        pl.BlockSpec((2 * D, D), lambda i: (0, 0)),
        pl.BlockSpec((1, D), lambda i: (0, 0)),
    ]
    return pl.pallas_call(
        body,
        grid=grid,
        in_specs=in_specs,
        out_specs=pl.BlockSpec((R, D), lambda i: (i, 0)),
        out_shape=jax.ShapeDtypeStruct((N, D), jnp.float32),
    )


_mlp0 = _make_mlp(final_relu=True, with_sd_inputs=True)
_mlp1 = _make_mlp(final_relu=False, with_sd_inputs=True)

_SCALE = 1.0 / np.sqrt(1.0 + EPS)


def kernel(x, edge_index, edge_attr, self_loop_index, self_loop_type,
           W_enc0, b_enc0, W1_0, b1_0, gamma0, beta0, W2_0, b2_0,
           W_enc1, b_enc1, W1_1, b1_1, gamma1, beta1, W2_1, b2_1):
    dst = edge_index[0]
    src = edge_index[1]
    sl_row = ((jnp.arange(DE) == self_loop_index).astype(jnp.float32)
              * jnp.asarray(self_loop_type, jnp.float32))

    def fold(W1, b1, gamma, beta):
        g = gamma * _SCALE
        return W1 * g[None, :], b1 * g + beta

    W1f0, b1f0 = fold(W1_0, b1_0, gamma0, beta0)
    A0 = W1f0
    B0 = W_enc0 @ W1f0
    v0 = (b_enc0 @ W1f0)[None, :]
    u0 = ((sl_row @ W_enc0 + b_enc0) @ W1f0 + b1f0)[None, :]

    W1f1, b1f1 = fold(W1_1, b1_1, gamma1, beta1)
    A1 = W1f1[:D]
    Wb = W1f1[D:]
    B1 = W_enc1 @ Wb
    v1 = (b_enc1 @ Wb)[None, :]
    u1 = ((sl_row @ W_enc1 + b_enc1) @ Wb + b1f1)[None, :]

    aggx, S, deg = _spmm_sd(x, src, dst, edge_attr)
    h0 = _mlp0(aggx, x, S, deg, A0, B0, v0, u0, W2_0, b2_0[None, :])
    (aggh,) = _spmm(h0, src, dst, edge_attr)
    h1 = _mlp1(aggh, h0, S, deg, A1, B1, v1, u1, W2_1, b2_1[None, :])
    return h1


# speedup vs baseline: 11.5086x; 1.7737x over previous
import functools

import jax
import jax.numpy as jnp
import numpy as np
from jax import lax
from jax.experimental import pallas as pl
from jax.experimental.pallas import tpu as pltpu
from jax.experimental.pallas import tpu_sc as plsc

N = 10000
E = 320000
D = 128
DE = 16
EPS = 1e-05

NC = 2
NS = 16
NW = NC * NS
EPW = E // NW
K = 80
NCHUNK = EPW // K
RO = 624
ZR = 16
TAIL = N - NS * RO

_MESH = plsc.VectorSubcoreMesh(core_axis_name="c", subcore_axis_name="s")


def _make_spmm(with_sd: bool):
    out_type = [jax.ShapeDtypeStruct((NC, N, D), jnp.float32)]
    if with_sd:
        out_type += [jax.ShapeDtypeStruct((NC, N, DE), jnp.float32),
                     jax.ShapeDtypeStruct((NC, N, DE), jnp.float32)]
    scratch = [
        [pltpu.VMEM((2, K), jnp.int32)] * 2,
        [pltpu.VMEM((K, D), jnp.float32)] * 2,
        pltpu.VMEM((ZR, D), jnp.float32),
        pltpu.VMEM_SHARED((N, D), jnp.float32),
        [pltpu.SemaphoreType.DMA] * 2,
        [pltpu.SemaphoreType.DMA] * 2,
    ]
    if with_sd:
        scratch += [
            [pltpu.VMEM((K, DE), jnp.float32)] * 2,
            pltpu.VMEM((K, DE), jnp.float32),
            pltpu.VMEM((ZR, DE), jnp.float32),
            pltpu.VMEM_SHARED((N, DE), jnp.float32),
            pltpu.VMEM_SHARED((N, DE), jnp.float32),
        ]

    def body(h_hbm, ei_hbm, ea_hbm, *refs):
        if with_sd:
            (agg_out, s_out, deg_out, eiv, rows, zbuf, agg_sh, sem_i, sem_g,
             eav, ones, zbuf16, s_sh, deg_sh) = refs
        else:
            agg_out, eiv, rows, zbuf, agg_sh, sem_i, sem_g = refs
        cid = lax.axis_index("c")
        sid = lax.axis_index("s")

        zv = jnp.zeros((16,), jnp.float32)

        @pl.loop(0, ZR)
        def _(i):
            @pl.loop(0, D // 16)
            def _(j):
                zbuf[i, pl.ds(j * 16, 16)] = zv

        if with_sd:
            ov = jnp.ones((16,), jnp.float32)

            @pl.loop(0, ZR)
            def _(i):
                zbuf16[i, pl.ds(0, 16)] = zv

            @pl.loop(0, K)
            def _(i):
                ones[i, pl.ds(0, 16)] = ov

        rbase = sid * RO
        for r in range(RO // ZR):
            pltpu.sync_copy(zbuf, agg_sh.at[pl.ds(rbase + r * ZR, ZR)])
            if with_sd:
                pltpu.sync_copy(zbuf16, s_sh.at[pl.ds(rbase + r * ZR, ZR)])
                pltpu.sync_copy(zbuf16, deg_sh.at[pl.ds(rbase + r * ZR, ZR)])

        @pl.when(sid == NS - 1)
        def _():
            pltpu.sync_copy(zbuf.at[pl.ds(0, TAIL)], agg_sh.at[pl.ds(NS * RO, TAIL)])
            if with_sd:
                pltpu.sync_copy(zbuf16.at[pl.ds(0, TAIL)], s_sh.at[pl.ds(NS * RO, TAIL)])
                pltpu.sync_copy(zbuf16.at[pl.ds(0, TAIL)], deg_sh.at[pl.ds(NS * RO, TAIL)])

        plsc.subcore_barrier()

        ebase = (sid * NC + cid) * EPW

        def start_inputs(c, b):
            off = ebase + c * K
            pltpu.async_copy(ei_hbm.at[:, pl.ds(off, K)], eiv[b], sem_i[b])
            if with_sd:
                pltpu.async_copy(ea_hbm.at[pl.ds(off, K)], eav[b], sem_i[b])

        def wait_inputs(b):
            pltpu.make_async_copy(ei_hbm.at[:, pl.ds(0, K)], eiv[b], sem_i[b]).wait()
            if with_sd:
                pltpu.make_async_copy(ea_hbm.at[pl.ds(0, K)], eav[b], sem_i[b]).wait()

        def start_gather(b):
            pltpu.async_copy(h_hbm.at[eiv[b].at[1]], rows[b], sem_g[b])

        def wait_gather(b):
            pltpu.make_async_copy(h_hbm.at[eiv[b].at[1]], rows[b], sem_g[b]).wait()

        def scatter(b):
            dstr = eiv[b].at[0]
            pltpu.sync_copy(rows[b], agg_sh.at[dstr], add=True)
            if with_sd:
                pltpu.sync_copy(eav[b], s_sh.at[dstr], add=True)
                pltpu.sync_copy(ones, deg_sh.at[dstr], add=True)

        start_inputs(0, 0)
        start_inputs(1, 1)
        wait_inputs(0)
        start_gather(0)

        @pl.loop(0, NCHUNK - 1, step=2)
        def _(j):
            for b in (0, 1):
                c = j + b
                wait_inputs(1 - b)
                start_gather(1 - b)
                wait_gather(b)
                scatter(b)

                @pl.when(c + 2 < NCHUNK)
                def _():
                    start_inputs(c + 2, b)

        wait_gather((NCHUNK - 1) % 2)
        scatter((NCHUNK - 1) % 2)

        plsc.subcore_barrier()
        pltpu.sync_copy(agg_sh.at[pl.ds(rbase, RO)],
                        agg_out.at[cid, pl.ds(rbase, RO)])
        if with_sd:
            pltpu.sync_copy(s_sh.at[pl.ds(rbase, RO)],
                            s_out.at[cid, pl.ds(rbase, RO)])
            pltpu.sync_copy(deg_sh.at[pl.ds(rbase, RO)],
                            deg_out.at[cid, pl.ds(rbase, RO)])

        @pl.when(sid == NS - 1)
        def _():
            pltpu.sync_copy(agg_sh.at[pl.ds(NS * RO, TAIL)],
                            agg_out.at[cid, pl.ds(NS * RO, TAIL)])
            if with_sd:
                pltpu.sync_copy(s_sh.at[pl.ds(NS * RO, TAIL)],
                                s_out.at[cid, pl.ds(NS * RO, TAIL)])
                pltpu.sync_copy(deg_sh.at[pl.ds(NS * RO, TAIL)],
                                deg_out.at[cid, pl.ds(NS * RO, TAIL)])

    return functools.partial(
        pl.kernel, mesh=_MESH, out_type=tuple(out_type), scratch_types=scratch,
        compiler_params=pltpu.CompilerParams(use_tc_tiling_on_sc=False))(body)


_spmm_sd = _make_spmm(True)
_spmm = _make_spmm(False)


def _make_mlp(final_relu: bool, with_sd_inputs: bool):
    R = 2000

    def body(agg_ref, h_ref, s_ref, d_ref, a_ref, b_ref, v_ref, u_ref,
             w2_ref, b2_ref, o_ref):
        z = agg_ref[0] + agg_ref[1] + h_ref[...]
        sarr = s_ref[0] + s_ref[1]
        darr = d_ref[0] + d_ref[1]
        dcol = darr[:, :1]
        pre = (jnp.dot(z, a_ref[...], preferred_element_type=jnp.float32)
               + jnp.dot(sarr, b_ref[...], preferred_element_type=jnp.float32)
               + dcol * v_ref[...] + u_ref[...])
        t = jnp.maximum(pre, 0.0)
        out = jnp.dot(t, w2_ref[...], preferred_element_type=jnp.float32) + b2_ref[...]
        if final_relu:
            out = jnp.maximum(out, 0.0)
        o_ref[...] = out

    grid = (N // R,)
    in_specs = [
        pl.BlockSpec((NC, R, D), lambda i: (0, i, 0)),
        pl.BlockSpec((R, D), lambda i: (i, 0)),
        pl.BlockSpec((NC, R, DE), lambda i: (0, i, 0)),
        pl.BlockSpec((NC, R, DE), lambda i: (0, i, 0)),
        pl.BlockSpec((D, 2 * D), lambda i: (0, 0)),
        pl.BlockSpec((DE, 2 * D), lambda i: (0, 0)),
        pl.BlockSpec((1, 2 * D), lambda i: (0, 0)),
        pl.BlockSpec((1, 2 * D), lambda i: (0, 0)),
        pl.BlockSpec((2 * D, D), lambda i: (0, 0)),
        pl.BlockSpec((1, D), lambda i: (0, 0)),
    ]
    return pl.pallas_call(
        body,
        grid=grid,
        in_specs=in_specs,
        out_specs=pl.BlockSpec((R, D), lambda i: (i, 0)),
        out_shape=jax.ShapeDtypeStruct((N, D), jnp.float32),
    )


_mlp0 = _make_mlp(final_relu=True, with_sd_inputs=True)
_mlp1 = _make_mlp(final_relu=False, with_sd_inputs=True)

_SCALE = 1.0 / np.sqrt(1.0 + EPS)


def kernel(x, edge_index, edge_attr, self_loop_index, self_loop_type,
           W_enc0, b_enc0, W1_0, b1_0, gamma0, beta0, W2_0, b2_0,
           W_enc1, b_enc1, W1_1, b1_1, gamma1, beta1, W2_1, b2_1):
    sl_row = ((jnp.arange(DE) == self_loop_index).astype(jnp.float32)
              * jnp.asarray(self_loop_type, jnp.float32))

    def fold(W1, b1, gamma, beta):
        g = gamma * _SCALE
        return W1 * g[None, :], b1 * g + beta

    W1f0, b1f0 = fold(W1_0, b1_0, gamma0, beta0)
    A0 = W1f0
    B0 = W_enc0 @ W1f0
    v0 = (b_enc0 @ W1f0)[None, :]
    u0 = ((sl_row @ W_enc0 + b_enc0) @ W1f0 + b1f0)[None, :]

    W1f1, b1f1 = fold(W1_1, b1_1, gamma1, beta1)
    A1 = W1f1[:D]
    Wb = W1f1[D:]
    B1 = W_enc1 @ Wb
    v1 = (b_enc1 @ Wb)[None, :]
    u1 = ((sl_row @ W_enc1 + b_enc1) @ Wb + b1f1)[None, :]

    aggx, S, deg = _spmm_sd(x, edge_index, edge_attr)
    h0 = _mlp0(aggx, x, S, deg, A0, B0, v0, u0, W2_0, b2_0[None, :])
    (aggh,) = _spmm(h0, edge_index, edge_attr)
    h1 = _mlp1(aggh, h0, S, deg, A1, B1, v1, u1, W2_1, b2_1[None, :])
    return h1


# speedup vs baseline: 12.8191x; 1.1139x over previous
import functools

import jax
import jax.numpy as jnp
import numpy as np
from jax import lax
from jax.experimental import pallas as pl
from jax.experimental.pallas import tpu as pltpu
from jax.experimental.pallas import tpu_sc as plsc

N = 10000
E = 320000
D = 128
DE = 16
EPS = 1e-05

NC = 2
NS = 16
NW = NC * NS
EPW = E // NW
K = 128
NFULL = EPW // K
TE = EPW - NFULL * K
RO = 624
ZR = 48
TAIL = N - NS * RO

_MESH = plsc.VectorSubcoreMesh(core_axis_name="c", subcore_axis_name="s")
_SC_PARAMS = pltpu.CompilerParams(use_tc_tiling_on_sc=False)


def _zero_fill(zbuf, width):
    zv = jnp.zeros((16,), jnp.float32)

    @pl.loop(0, ZR)
    def _(i):
        @pl.loop(0, width // 16)
        def _(j):
            zbuf[i, pl.ds(j * 16, 16)] = zv


def _zero_shared(sid, zbuf, sh):
    rbase = sid * RO
    for r in range(RO // ZR):
        pltpu.sync_copy(zbuf, sh.at[pl.ds(rbase + r * ZR, ZR)])

    @pl.when(sid == NS - 1)
    def _():
        pltpu.sync_copy(zbuf.at[pl.ds(0, TAIL)], sh.at[pl.ds(NS * RO, TAIL)])


def _write_out(sid, cid, sh, out):
    rbase = sid * RO
    pltpu.sync_copy(sh.at[pl.ds(rbase, RO)], out.at[cid, pl.ds(rbase, RO)])

    @pl.when(sid == NS - 1)
    def _():
        pltpu.sync_copy(sh.at[pl.ds(NS * RO, TAIL)], out.at[cid, pl.ds(NS * RO, TAIL)])


def _make_spmm():
    scratch = [
        [pltpu.VMEM((2, K), jnp.int32)] * 2,
        [pltpu.VMEM((K, D), jnp.float32)] * 2,
        pltpu.VMEM((2, TE), jnp.int32),
        pltpu.VMEM((TE, D), jnp.float32),
        pltpu.VMEM((ZR, D), jnp.float32),
        pltpu.VMEM_SHARED((N, D), jnp.float32),
        [pltpu.SemaphoreType.DMA] * 2,
        [pltpu.SemaphoreType.DMA] * 2,
        pltpu.SemaphoreType.DMA,
    ]

    def body(h_hbm, ei_hbm, agg_out, eiv, rows, eiv_t, rows_t, zbuf, agg_sh,
             sem_i, sem_g, sem_t):
        cid = lax.axis_index("c")
        sid = lax.axis_index("s")
        _zero_fill(zbuf, D)
        _zero_shared(sid, zbuf, agg_sh)
        plsc.subcore_barrier()

        ebase = (sid * NC + cid) * EPW

        def start_inputs(c, b):
            pltpu.async_copy(ei_hbm.at[:, pl.ds(ebase + c * K, K)], eiv[b], sem_i[b])

        def wait_inputs(b):
            pltpu.make_async_copy(ei_hbm.at[:, pl.ds(0, K)], eiv[b], sem_i[b]).wait()

        def start_gather(b):
            pltpu.async_copy(h_hbm.at[eiv[b].at[1]], rows[b], sem_g[b])

        def wait_gather(b):
            pltpu.make_async_copy(h_hbm.at[eiv[b].at[1]], rows[b], sem_g[b]).wait()

        def scatter(b):
            pltpu.sync_copy(rows[b], agg_sh.at[eiv[b].at[0]], add=True)

        start_inputs(0, 0)
        start_inputs(1, 1)
        wait_inputs(0)
        start_gather(0)

        @pl.loop(0, NFULL - 2, step=2)
        def _(j):
            for b in (0, 1):
                c = j + b
                wait_inputs(1 - b)
                start_gather(1 - b)
                wait_gather(b)
                scatter(b)
                start_inputs(c + 2, b)

        wait_inputs(1)
        start_gather(1)
        wait_gather(0)
        scatter(0)
        wait_gather(1)
        scatter(1)

        toff = ebase + NFULL * K
        pltpu.sync_copy(ei_hbm.at[:, pl.ds(toff, TE)], eiv_t)
        pltpu.async_copy(h_hbm.at[eiv_t.at[1]], rows_t, sem_t).wait()
        pltpu.sync_copy(rows_t, agg_sh.at[eiv_t.at[0]], add=True)

        plsc.subcore_barrier()
        _write_out(sid, cid, agg_sh, agg_out)

    return functools.partial(
        pl.kernel, mesh=_MESH,
        out_type=(jax.ShapeDtypeStruct((NC, N, D), jnp.float32),),
        scratch_types=scratch, compiler_params=_SC_PARAMS)(body)


def _make_sdeg():
    scratch = [
        [pltpu.VMEM((2, K), jnp.int32)] * 2,
        [pltpu.VMEM((K, DE), jnp.float32)] * 2,
        pltpu.VMEM((K, DE), jnp.float32),
        pltpu.VMEM((2, TE), jnp.int32),
        pltpu.VMEM((TE, DE), jnp.float32),
        pltpu.VMEM((TE, DE), jnp.float32),
        pltpu.VMEM((ZR, DE), jnp.float32),
        pltpu.VMEM_SHARED((N, DE), jnp.float32),
        pltpu.VMEM_SHARED((N, DE), jnp.float32),
        [pltpu.SemaphoreType.DMA] * 2,
    ]

    def body(ei_hbm, ea_hbm, s_out, deg_out, eiv, eav, ones, eiv_t, eav_t,
             ones_t, zbuf16, s_sh, deg_sh, sem_i):
        cid = lax.axis_index("c")
        sid = lax.axis_index("s")
        ov = jnp.ones((16,), jnp.float32)

        @pl.loop(0, K)
        def _(i):
            ones[i, pl.ds(0, 16)] = ov

        @pl.loop(0, TE)
        def _(i):
            ones_t[i, pl.ds(0, 16)] = ov

        _zero_fill(zbuf16, DE)
        _zero_shared(sid, zbuf16, s_sh)
        _zero_shared(sid, zbuf16, deg_sh)
        plsc.subcore_barrier()

        ebase = (sid * NC + cid) * EPW

        def start_inputs(c, b):
            off = ebase + c * K
            pltpu.async_copy(ei_hbm.at[:, pl.ds(off, K)], eiv[b], sem_i[b])
            pltpu.async_copy(ea_hbm.at[pl.ds(off, K)], eav[b], sem_i[b])

        def wait_inputs(b):
            pltpu.make_async_copy(ei_hbm.at[:, pl.ds(0, K)], eiv[b], sem_i[b]).wait()
            pltpu.make_async_copy(ea_hbm.at[pl.ds(0, K)], eav[b], sem_i[b]).wait()

        def scatter(b):
            dstr = eiv[b].at[0]
            pltpu.sync_copy(eav[b], s_sh.at[dstr], add=True)
            pltpu.sync_copy(ones, deg_sh.at[dstr], add=True)

        start_inputs(0, 0)
        start_inputs(1, 1)

        @pl.loop(0, NFULL - 2, step=2)
        def _(j):
            for b in (0, 1):
                c = j + b
                wait_inputs(b)
                scatter(b)
                start_inputs(c + 2, b)

        wait_inputs(0)
        scatter(0)
        wait_inputs(1)
        scatter(1)

        toff = ebase + NFULL * K
        pltpu.sync_copy(ei_hbm.at[:, pl.ds(toff, TE)], eiv_t)
        pltpu.sync_copy(ea_hbm.at[pl.ds(toff, TE)], eav_t)
        pltpu.sync_copy(eav_t, s_sh.at[eiv_t.at[0]], add=True)
        pltpu.sync_copy(ones_t, deg_sh.at[eiv_t.at[0]], add=True)

        plsc.subcore_barrier()
        _write_out(sid, cid, s_sh, s_out)
        _write_out(sid, cid, deg_sh, deg_out)

    return functools.partial(
        pl.kernel, mesh=_MESH,
        out_type=(jax.ShapeDtypeStruct((NC, N, DE), jnp.float32),
                  jax.ShapeDtypeStruct((NC, N, DE), jnp.float32)),
        scratch_types=scratch, compiler_params=_SC_PARAMS)(body)


_spmm = _make_spmm()
_sdeg = _make_sdeg()


def _make_mlp(final_relu: bool, with_sd_inputs: bool):
    R = 2000

    def body(agg_ref, h_ref, s_ref, d_ref, a_ref, b_ref, v_ref, u_ref,
             w2_ref, b2_ref, o_ref):
        z = agg_ref[0] + agg_ref[1] + h_ref[...]
        sarr = s_ref[0] + s_ref[1]
        darr = d_ref[0] + d_ref[1]
        dcol = darr[:, :1]
        pre = (jnp.dot(z, a_ref[...], preferred_element_type=jnp.float32)
               + jnp.dot(sarr, b_ref[...], preferred_element_type=jnp.float32)
               + dcol * v_ref[...] + u_ref[...])
        t = jnp.maximum(pre, 0.0)
        out = jnp.dot(t, w2_ref[...], preferred_element_type=jnp.float32) + b2_ref[...]
        if final_relu:
            out = jnp.maximum(out, 0.0)
        o_ref[...] = out

    grid = (N // R,)
    in_specs = [
        pl.BlockSpec((NC, R, D), lambda i: (0, i, 0)),
        pl.BlockSpec((R, D), lambda i: (i, 0)),
        pl.BlockSpec((NC, R, DE), lambda i: (0, i, 0)),
        pl.BlockSpec((NC, R, DE), lambda i: (0, i, 0)),
        pl.BlockSpec((D, 2 * D), lambda i: (0, 0)),
        pl.BlockSpec((DE, 2 * D), lambda i: (0, 0)),
        pl.BlockSpec((1, 2 * D), lambda i: (0, 0)),
        pl.BlockSpec((1, 2 * D), lambda i: (0, 0)),
        pl.BlockSpec((2 * D, D), lambda i: (0, 0)),
        pl.BlockSpec((1, D), lambda i: (0, 0)),
    ]
    return pl.pallas_call(
        body,
        grid=grid,
        in_specs=in_specs,
        out_specs=pl.BlockSpec((R, D), lambda i: (i, 0)),
        out_shape=jax.ShapeDtypeStruct((N, D), jnp.float32),
    )


_mlp0 = _make_mlp(final_relu=True, with_sd_inputs=True)
_mlp1 = _make_mlp(final_relu=False, with_sd_inputs=True)

_SCALE = 1.0 / np.sqrt(1.0 + EPS)


def kernel(x, edge_index, edge_attr, self_loop_index, self_loop_type,
           W_enc0, b_enc0, W1_0, b1_0, gamma0, beta0, W2_0, b2_0,
           W_enc1, b_enc1, W1_1, b1_1, gamma1, beta1, W2_1, b2_1):
    sl_row = ((jnp.arange(DE) == self_loop_index).astype(jnp.float32)
              * jnp.asarray(self_loop_type, jnp.float32))

    def fold(W1, b1, gamma, beta):
        g = gamma * _SCALE
        return W1 * g[None, :], b1 * g + beta

    W1f0, b1f0 = fold(W1_0, b1_0, gamma0, beta0)
    A0 = W1f0
    B0 = W_enc0 @ W1f0
    v0 = (b_enc0 @ W1f0)[None, :]
    u0 = ((sl_row @ W_enc0 + b_enc0) @ W1f0 + b1f0)[None, :]

    W1f1, b1f1 = fold(W1_1, b1_1, gamma1, beta1)
    A1 = W1f1[:D]
    Wb = W1f1[D:]
    B1 = W_enc1 @ Wb
    v1 = (b_enc1 @ Wb)[None, :]
    u1 = ((sl_row @ W_enc1 + b_enc1) @ Wb + b1f1)[None, :]

    (aggx,) = _spmm(x, edge_index)
    S, deg = _sdeg(edge_index, edge_attr)
    h0 = _mlp0(aggx, x, S, deg, A0, B0, v0, u0, W2_0, b2_0[None, :])
    (aggh,) = _spmm(h0, edge_index)
    h1 = _mlp1(aggh, h0, S, deg, A1, B1, v1, u1, W2_1, b2_1[None, :])
    return h1
